# baseline (speedup 1.0000x reference)
import functools

import jax
import jax.numpy as jnp
from jax import lax
from jax.experimental import pallas as pl
from jax.experimental.pallas import tpu as pltpu
from jax.experimental.pallas import tpu_sc as plsc


def _build_gather(batch, hist, V, D, b_per_w, chunk, nbuf, n_groups, NC):
    mesh = plsc.VectorSubcoreMesh(core_axis_name="c", subcore_axis_name="s")
    B = batch * hist

    rows_per_chunk = chunk // hist
    scratch = (
        [pltpu.VMEM((chunk,), jnp.int32) for _ in range(nbuf)]
        + [pltpu.VMEM((chunk, D), jnp.float32) for _ in range(nbuf)]
        + [pltpu.VMEM((rows_per_chunk, hist), jnp.int32) for _ in range(nbuf)]
        + [pltpu.SemaphoreType.DMA for _ in range(3 * nbuf)]
    )

    @functools.partial(
        pl.kernel,
        mesh=mesh,
        out_type=jax.ShapeDtypeStruct((batch, hist, D), jnp.float32),
        scratch_types=scratch,
        compiler_params=pltpu.CompilerParams(use_tc_tiling_on_sc=False),
    )
    def gather_kernel(x_hbm2, table_hbm, out_hbm3, *bufs):
        idx_v = bufs[:nbuf]
        rows_v = bufs[nbuf:2 * nbuf]
        xrow_v = bufs[2 * nbuf:3 * nbuf]
        isem = bufs[3 * nbuf:4 * nbuf]
        gsem = bufs[4 * nbuf:5 * nbuf]
        osem = bufs[5 * nbuf:6 * nbuf]

        wid = lax.axis_index("s") * NC + lax.axis_index("c")
        base = wid * b_per_w
        xrow0 = wid * (b_per_w // hist)

        col_starts = list(range(0, hist - 15, 16))
        if col_starts[-1] != hist - 16:
            col_starts.append(hist - 16)

        def start_idx(c, b):
            pltpu.async_copy(
                x_hbm2.at[pl.ds(xrow0 + c * rows_per_chunk, rows_per_chunk), :],
                xrow_v[b], isem[b])

        def wait_idx(b):
            pltpu.make_async_copy(
                x_hbm2.at[pl.ds(xrow0, rows_per_chunk), :],
                xrow_v[b], isem[b]).wait()

        def compact_idx(b):
            for r in range(rows_per_chunk):
                for c in col_starts:
                    idx_v[b][pl.ds(r * hist + c, 16)] = (
                        xrow_v[b][r, pl.ds(c, 16)])

        def start_gather(b):
            pltpu.async_copy(table_hbm.at[idx_v[b]], rows_v[b], gsem[b])

        def wait_gather(b):
            pltpu.make_async_copy(table_hbm.at[idx_v[b]], rows_v[b],
                                  gsem[b]).wait()

        def start_out(c, b):
            for r in range(rows_per_chunk):
                pltpu.async_copy(
                    rows_v[b].at[pl.ds(r * hist, hist), :],
                    out_hbm3.at[xrow0 + c * rows_per_chunk + r],
                    osem[b])

        def wait_out(b):
            pltpu.make_async_copy(table_hbm.at[pl.ds(0, chunk), :],
                                  rows_v[b], osem[b]).wait()

        start_idx(0, 0)

        def body(g, carry):
            c0 = g * nbuf
            for b in range(1, nbuf):
                start_idx(c0 + b, b)
            for b in range(nbuf):
                wait_idx(b)
                compact_idx(b)

                @pl.when(g > 0)
                def _():
                    wait_out(b)

                start_gather(b)
            for b in range(nbuf):
                wait_gather(b)
                start_out(c0 + b, b)
                if b == 0:
                    @pl.when(g < n_groups - 1)
                    def _():
                        start_idx(c0 + nbuf, 0)
            return carry

        lax.fori_loop(0, n_groups, body, 0)
        for b in range(nbuf):
            wait_out(b)

    return gather_kernel


def kernel(x, table):
    batch, hist = x.shape
    V, D = table.shape
    B = batch * hist

    info = plsc.get_sparse_core_info()
    NC, NS = info.num_cores, info.num_subcores
    NW = NC * NS
    b_per_w = B // NW
    chunk = 800
    nbuf = 4
    n_groups = b_per_w // (chunk * nbuf)

    return _build_gather(batch, hist, V, D, b_per_w, chunk, nbuf,
                         n_groups, NC)(x, table)
